# baseline (speedup 1.0000x reference)
import functools

import jax
import jax.numpy as jnp
from jax import lax
from jax.experimental import pallas as pl
from jax.experimental.pallas import tpu as pltpu
from jax.experimental.pallas import tpu_sc as plsc

_VOCAB1 = 1000001
_TSH_LEN = 1000448
_NC, _NS = 2, 16
_NW = _NC * _NS
_N_DOCS = 4096
_WPD = 200
_DOCS_PER_W = _N_DOCS // _NW
_IDX_PER_W = _DOCS_PER_W * _WPD
_CHUNK = 128
_NCHUNK = _IDX_PER_W // _CHUNK
_PIECE = 1952
_NPIECE = 32
_SEG = _PIECE * _NPIECE
_MAIN = _SEG * _NS
_TAIL = _VOCAB1 - _MAIN
_TAIL_VREGS = -(-_TAIL // 16)


def _body(idx_hbm, w_hbm, r_hbm, scal_hbm, out_hbm,
          t_sh, idx_v, vals_v, wbuf0, wbuf1, rbuf0, rbuf1, tbuf0, tbuf1,
          tmp_v, out_v, scal_v, sem_idx, si0, si1, so0, so1, gsem):
    sid = lax.axis_index("s")
    wid = sid * _NC + lax.axis_index("c")
    pltpu.async_copy(idx_hbm.at[wid], idx_v, sem_idx)
    pltpu.sync_copy(scal_hbm, scal_v)
    scal = scal_v[pl.ds(0, 16)]
    w_adj = scal[0]
    r_inv = scal[1]
    lane = lax.iota(jnp.int32, 16)

    si = (si0, si1)
    so = (so0, so1)
    wb = (wbuf0, wbuf1)
    rb = (rbuf0, rbuf1)
    tb = (tbuf0, tbuf1)

    def fire_in(p):
        slot = p % 2
        off = sid * _SEG + p * _PIECE
        pltpu.async_copy(w_hbm.at[pl.ds(off, _PIECE)], wb[slot], si[slot])
        pltpu.async_copy(r_hbm.at[pl.ds(off, _PIECE)], rb[slot], si[slot])

    def wait_in(p):
        slot = p % 2
        off = sid * _SEG + p * _PIECE
        pltpu.make_async_copy(w_hbm.at[pl.ds(off, _PIECE)], wb[slot],
                              si[slot]).wait()
        pltpu.make_async_copy(r_hbm.at[pl.ds(off, _PIECE)], rb[slot],
                              si[slot]).wait()

    def wait_out(p):
        slot = p % 2
        off = sid * _SEG + p * _PIECE
        pltpu.make_async_copy(tb[slot], t_sh.at[pl.ds(off, _PIECE)],
                              so[slot]).wait()

    fire_in(0)
    for p in range(_NPIECE):
        slot = p % 2
        if p + 1 < _NPIECE:
            fire_in(p + 1)
        wait_in(p)
        if p >= 2:
            wait_out(p - 2)

        def piece_vreg(i, carry):
            for u in range(2):
                s = pl.ds(i * 32 + u * 16, 16)
                tb[slot][s] = (wb[slot][s] + w_adj) * rb[slot][s] * r_inv
            return carry

        lax.fori_loop(0, _PIECE // 32, piece_vreg, 0)
        if p == 0:
            @pl.when(sid == 0)
            def _():
                first = tbuf0[pl.ds(0, 16)]
                tbuf0[pl.ds(0, 16)] = jnp.where(
                    lane == 0, jnp.float32(0.0), first)
        off = sid * _SEG + p * _PIECE
        pltpu.async_copy(tb[slot], t_sh.at[pl.ds(off, _PIECE)], so[slot])
    wait_out(_NPIECE - 2)
    wait_out(_NPIECE - 1)

    @pl.when(sid == _NS - 1)
    def _():
        pltpu.sync_copy(w_hbm.at[pl.ds(_MAIN, _TAIL)],
                        wbuf0.at[pl.ds(0, _TAIL)])
        pltpu.sync_copy(r_hbm.at[pl.ds(_MAIN, _TAIL)],
                        rbuf0.at[pl.ds(0, _TAIL)])
        for i in range(_TAIL_VREGS):
            s = pl.ds(i * 16, 16)
            tbuf0[s] = (wbuf0[s] + w_adj) * rbuf0[s] * r_inv
        pltpu.sync_copy(tbuf0.at[pl.ds(0, _TAIL)],
                        t_sh.at[pl.ds(_MAIN, _TAIL)])

    plsc.subcore_barrier()
    pltpu.make_async_copy(idx_hbm.at[wid], idx_v, sem_idx).wait()

    _CPG = _NCHUNK // (_DOCS_PER_W // 16)

    def fire(c, carry):
        pltpu.async_copy(t_sh.at[idx_v.at[c]],
                         vals_v.at[pl.ds(c * _CHUNK, _CHUNK)],
                         gsem.at[c // _CPG])
        return carry

    lax.fori_loop(0, _NCHUNK, fire, 0)

    tail_mask = lane < 8
    lanes16 = lane * 16

    def drain(c, carry):
        pltpu.make_async_copy(t_sh.at[idx_v.at[c]],
                              vals_v.at[pl.ds(c * _CHUNK, _CHUNK)],
                              gsem.at[c // _CPG]).wait()
        return carry

    def group(g, carry):
        lax.fori_loop(g * _CPG, (g + 1) * _CPG, drain, 0)
        for l in range(16):
            base = g * 3200 + l * _WPD
            acc = vals_v[pl.ds(base, 16)]
            for j in range(1, 12):
                acc = acc + vals_v[pl.ds(base + j * 16, 16)]
            tail = vals_v[pl.ds(base + 192, 16)]
            acc = acc + jnp.where(tail_mask, tail, jnp.float32(0.0))
            tmp_v[pl.ds(l * 16, 16)] = acc
        tot = plsc.load_gather(tmp_v, [lanes16])
        for k in range(1, 16):
            tot = tot + plsc.load_gather(tmp_v, [lanes16 + k])
        out_v[pl.ds(g * 16, 16)] = tot
        return carry

    lax.fori_loop(0, _DOCS_PER_W // 16, group, 0)
    pltpu.sync_copy(out_v, out_hbm.at[pl.ds(wid * _DOCS_PER_W, _DOCS_PER_W)])


_sc_kernel = functools.partial(
    pl.kernel,
    out_type=jax.ShapeDtypeStruct((_N_DOCS,), jnp.float32),
    mesh=plsc.VectorSubcoreMesh(
        core_axis_name="c", subcore_axis_name="s",
        num_cores=_NC, num_subcores=_NS),
    scratch_types=[
        pltpu.VMEM_SHARED((_TSH_LEN,), jnp.float32),
        pltpu.VMEM((_NCHUNK, _CHUNK), jnp.int32),
        pltpu.VMEM((_IDX_PER_W + 16,), jnp.float32),
        pltpu.VMEM((_PIECE,), jnp.float32),
        pltpu.VMEM((_PIECE,), jnp.float32),
        pltpu.VMEM((_PIECE,), jnp.float32),
        pltpu.VMEM((_PIECE,), jnp.float32),
        pltpu.VMEM((_PIECE,), jnp.float32),
        pltpu.VMEM((_PIECE,), jnp.float32),
        pltpu.VMEM((256,), jnp.float32),
        pltpu.VMEM((_DOCS_PER_W,), jnp.float32),
        pltpu.VMEM((16,), jnp.float32),
    ] + [pltpu.SemaphoreType.DMA] * 5 + [pltpu.SemaphoreType.DMA((8,))],
    compiler_params=pltpu.CompilerParams(needs_layout_passes=False),
)(_body)


@jax.jit
def kernel(feat_idx, w_weight, r_weight, w_adj, r_adj):
    scal = jnp.stack([w_adj, 1.0 / r_adj] + [w_adj] * 14).astype(jnp.float32)
    idx3 = feat_idx.reshape(_NW, _NCHUNK, _CHUNK)
    return _sc_kernel(idx3, w_weight, r_weight, scal)


# speedup vs baseline: 1.0116x; 1.0116x over previous
import functools

import jax
import jax.numpy as jnp
from jax import lax
from jax.experimental import pallas as pl
from jax.experimental.pallas import tpu as pltpu
from jax.experimental.pallas import tpu_sc as plsc

_VOCAB1 = 1000001
_TSH_LEN = 1000448
_NC, _NS = 2, 16
_NW = _NC * _NS
_N_DOCS = 4096
_WPD = 200
_DOCS_PER_W = _N_DOCS // _NW
_IDX_PER_W = _DOCS_PER_W * _WPD
_CHUNK = 128
_NCHUNK = _IDX_PER_W // _CHUNK
_PIECE = 1952
_NPIECE = 32
_SEG = _PIECE * _NPIECE
_MAIN = _SEG * _NS
_TAIL = _VOCAB1 - _MAIN
_TAIL_VREGS = -(-_TAIL // 16)


def _body(idx_hbm, w_hbm, r_hbm, scal_hbm, out_hbm,
          t_sh, idx_v, vals_v, wbuf0, wbuf1, rbuf0, rbuf1, tbuf0, tbuf1,
          tmp_v, out_v, scal_v, sem_idx, si0, si1, so0, so1, gsem):
    sid = lax.axis_index("s")
    wid = sid * _NC + lax.axis_index("c")
    pltpu.async_copy(idx_hbm.at[wid], idx_v, sem_idx)

    si = (si0, si1)
    so = (so0, so1)
    wb = (wbuf0, wbuf1)
    rb = (rbuf0, rbuf1)
    tb = (tbuf0, tbuf1)

    def fire_in(p):
        slot = p % 2
        off = sid * _SEG + p * _PIECE
        pltpu.async_copy(w_hbm.at[pl.ds(off, _PIECE)], wb[slot], si[slot])
        pltpu.async_copy(r_hbm.at[pl.ds(off, _PIECE)], rb[slot], si[slot])

    def wait_in(p):
        slot = p % 2
        off = sid * _SEG + p * _PIECE
        pltpu.make_async_copy(w_hbm.at[pl.ds(off, _PIECE)], wb[slot],
                              si[slot]).wait()
        pltpu.make_async_copy(r_hbm.at[pl.ds(off, _PIECE)], rb[slot],
                              si[slot]).wait()

    def wait_out(p):
        slot = p % 2
        off = sid * _SEG + p * _PIECE
        pltpu.make_async_copy(tb[slot], t_sh.at[pl.ds(off, _PIECE)],
                              so[slot]).wait()

    fire_in(0)
    fire_in(1)
    pltpu.sync_copy(scal_hbm, scal_v)
    scal = scal_v[pl.ds(0, 16)]
    w_adj = scal[0]
    r_inv = scal[1]
    lane = lax.iota(jnp.int32, 16)
    for p in range(_NPIECE):
        slot = p % 2
        wait_in(p)
        if p >= 2:
            wait_out(p - 2)

        def piece_vreg(i, carry):
            for u in range(2):
                s = pl.ds(i * 32 + u * 16, 16)
                tb[slot][s] = (wb[slot][s] + w_adj) * rb[slot][s] * r_inv
            return carry

        lax.fori_loop(0, _PIECE // 32, piece_vreg, 0)
        if p == 0:
            @pl.when(sid == 0)
            def _():
                first = tbuf0[pl.ds(0, 16)]
                tbuf0[pl.ds(0, 16)] = jnp.where(
                    lane == 0, jnp.float32(0.0), first)
        off = sid * _SEG + p * _PIECE
        pltpu.async_copy(tb[slot], t_sh.at[pl.ds(off, _PIECE)], so[slot])
        if p + 2 < _NPIECE:
            fire_in(p + 2)
    wait_out(_NPIECE - 2)
    wait_out(_NPIECE - 1)

    @pl.when(sid == _NS - 1)
    def _():
        pltpu.sync_copy(w_hbm.at[pl.ds(_MAIN, _TAIL)],
                        wbuf0.at[pl.ds(0, _TAIL)])
        pltpu.sync_copy(r_hbm.at[pl.ds(_MAIN, _TAIL)],
                        rbuf0.at[pl.ds(0, _TAIL)])
        for i in range(_TAIL_VREGS):
            s = pl.ds(i * 16, 16)
            tbuf0[s] = (wbuf0[s] + w_adj) * rbuf0[s] * r_inv
        pltpu.sync_copy(tbuf0.at[pl.ds(0, _TAIL)],
                        t_sh.at[pl.ds(_MAIN, _TAIL)])

    plsc.subcore_barrier()
    pltpu.make_async_copy(idx_hbm.at[wid], idx_v, sem_idx).wait()

    _CPG = _NCHUNK // (_DOCS_PER_W // 16)

    def fire(c, carry):
        pltpu.async_copy(t_sh.at[idx_v.at[c]],
                         vals_v.at[pl.ds(c * _CHUNK, _CHUNK)],
                         gsem.at[c // _CPG])
        return carry

    lax.fori_loop(0, _NCHUNK, fire, 0)

    tail_mask = lane < 8
    lanes16 = lane * 16

    def drain(c, carry):
        pltpu.make_async_copy(t_sh.at[idx_v.at[c]],
                              vals_v.at[pl.ds(c * _CHUNK, _CHUNK)],
                              gsem.at[c // _CPG]).wait()
        return carry

    def group(g, carry):
        lax.fori_loop(g * _CPG, (g + 1) * _CPG, drain, 0)
        for l in range(16):
            base = g * 3200 + l * _WPD
            acc = vals_v[pl.ds(base, 16)]
            for j in range(1, 12):
                acc = acc + vals_v[pl.ds(base + j * 16, 16)]
            tail = vals_v[pl.ds(base + 192, 16)]
            acc = acc + jnp.where(tail_mask, tail, jnp.float32(0.0))
            tmp_v[pl.ds(l * 16, 16)] = acc
        tot = plsc.load_gather(tmp_v, [lanes16])
        for k in range(1, 16):
            tot = tot + plsc.load_gather(tmp_v, [lanes16 + k])
        out_v[pl.ds(g * 16, 16)] = tot
        return carry

    lax.fori_loop(0, _DOCS_PER_W // 16, group, 0)
    pltpu.sync_copy(out_v, out_hbm.at[pl.ds(wid * _DOCS_PER_W, _DOCS_PER_W)])


_sc_kernel = functools.partial(
    pl.kernel,
    out_type=jax.ShapeDtypeStruct((_N_DOCS,), jnp.float32),
    mesh=plsc.VectorSubcoreMesh(
        core_axis_name="c", subcore_axis_name="s",
        num_cores=_NC, num_subcores=_NS),
    scratch_types=[
        pltpu.VMEM_SHARED((_TSH_LEN,), jnp.float32),
        pltpu.VMEM((_NCHUNK, _CHUNK), jnp.int32),
        pltpu.VMEM((_IDX_PER_W + 16,), jnp.float32),
        pltpu.VMEM((_PIECE,), jnp.float32),
        pltpu.VMEM((_PIECE,), jnp.float32),
        pltpu.VMEM((_PIECE,), jnp.float32),
        pltpu.VMEM((_PIECE,), jnp.float32),
        pltpu.VMEM((_PIECE,), jnp.float32),
        pltpu.VMEM((_PIECE,), jnp.float32),
        pltpu.VMEM((256,), jnp.float32),
        pltpu.VMEM((_DOCS_PER_W,), jnp.float32),
        pltpu.VMEM((16,), jnp.float32),
    ] + [pltpu.SemaphoreType.DMA] * 5 + [pltpu.SemaphoreType.DMA((8,))],
    compiler_params=pltpu.CompilerParams(needs_layout_passes=False),
)(_body)


@jax.jit
def kernel(feat_idx, w_weight, r_weight, w_adj, r_adj):
    scal = jnp.stack([w_adj, 1.0 / r_adj] + [w_adj] * 14).astype(jnp.float32)
    idx3 = feat_idx.reshape(_NW, _NCHUNK, _CHUNK)
    return _sc_kernel(idx3, w_weight, r_weight, scal)
